# Initial kernel scaffold; baseline (speedup 1.0000x reference)
import jax
import jax.numpy as jnp
from jax.experimental import pallas as pl


def kernel(x, edge_index, W1, b1, W2, b2):
    raise NotImplementedError("write your pallas kernel here")


# speedup vs baseline: 9.5476x; 9.5476x over previous
import functools

import jax
import jax.numpy as jnp
from jax import lax
from jax.experimental import pallas as pl
from jax.experimental.pallas import tpu as pltpu
from jax.experimental.pallas import tpu_sc as plsc

N = 10000
E = 160000
D_IN = 256
D_HID = 512

N_PAD = 10240
N_PAD_ROWS = N_PAD - N
LANES = 128
N_SUBCORES = 16
N_CORES = 2
BPT = -(-E // (N_SUBCORES * LANES * 8)) * 8
E_BATCHES = BPT * N_SUBCORES
E_PAD = E_BATCHES * LANES
RPT = N_PAD // N_SUBCORES

_MESH = plsc.VectorSubcoreMesh(
    core_axis_name="c", subcore_axis_name="s",
    num_cores=N_CORES, num_subcores=N_SUBCORES)


def _rsqrt16(x):
  i = lax.bitcast_convert_type(x, jnp.int32)
  i = 0x5F3759DF - lax.shift_right_arithmetic(i, 1)
  y = lax.bitcast_convert_type(i, jnp.float32)
  for _ in range(3):
    y = y * (1.5 - 0.5 * x * y * y)
  return y


def _k1_body(srcp, dstp, dis_hbm, s_hbm,
             deg_sp, t_sp, dis_sp, sidx, didx, ones_v, vals_v, buf_a, buf_b,
             sem):
  c = lax.axis_index("c")
  sid = lax.axis_index("s")
  erow0 = sid * BPT
  r0 = sid * RPT

  pltpu.sync_copy(dstp.at[pl.ds(erow0, BPT), :], didx)

  @pl.loop(0, RPT // 16)
  def _zero(i):
    buf_a[pl.ds(i * 16, 16)] = jnp.zeros((16,), jnp.float32)

  pltpu.sync_copy(buf_a, deg_sp.at[pl.ds(r0, RPT)])
  pltpu.sync_copy(buf_a, t_sp.at[pl.ds(r0, RPT)])
  for i in range(LANES // 16):
    ones_v[pl.ds(i * 16, 16)] = jnp.ones((16,), jnp.float32)
  plsc.subcore_barrier()

  @pl.loop(0, BPT)
  def _deg(b):
    pltpu.sync_copy(ones_v, deg_sp.at[didx.at[b]], add=True)

  plsc.subcore_barrier()

  pltpu.sync_copy(deg_sp.at[pl.ds(r0, RPT)], buf_a)

  @pl.loop(0, RPT // 16)
  def _dis(i):
    d = buf_a[pl.ds(i * 16, 16)] + 1.0
    buf_b[pl.ds(i * 16, 16)] = _rsqrt16(d)

  pltpu.sync_copy(buf_b, dis_sp.at[pl.ds(r0, RPT)])

  @pl.when(c == 0)
  def _():
    pltpu.sync_copy(buf_b, dis_hbm.at[pl.ds(r0, RPT)])

  plsc.subcore_barrier()

  pltpu.sync_copy(srcp.at[pl.ds(erow0, BPT), :], sidx)

  @pl.loop(0, BPT)
  def _t(b):
    pltpu.async_copy(dis_sp.at[sidx.at[b]], vals_v, sem).wait()
    pltpu.sync_copy(vals_v, t_sp.at[didx.at[b]], add=True)

  plsc.subcore_barrier()

  pltpu.sync_copy(t_sp.at[pl.ds(r0, RPT)], buf_a)

  @pl.loop(0, RPT // 16)
  def _s(i):
    d = buf_b[pl.ds(i * 16, 16)]
    buf_a[pl.ds(i * 16, 16)] = d * (buf_a[pl.ds(i * 16, 16)] + d)

  @pl.when(c == 0)
  def _():
    pltpu.sync_copy(buf_a, s_hbm.at[pl.ds(r0, RPT)])


_k1 = pl.kernel(
    _k1_body,
    out_type=[jax.ShapeDtypeStruct((N_PAD,), jnp.float32),
              jax.ShapeDtypeStruct((N_PAD,), jnp.float32)],
    mesh=_MESH,
    scratch_types=[
        pltpu.VMEM_SHARED((N_PAD,), jnp.float32),
        pltpu.VMEM_SHARED((N_PAD,), jnp.float32),
        pltpu.VMEM_SHARED((N_PAD,), jnp.float32),
        pltpu.VMEM((BPT, LANES), jnp.int32),
        pltpu.VMEM((BPT, LANES), jnp.int32),
        pltpu.VMEM((LANES,), jnp.float32),
        pltpu.VMEM((LANES,), jnp.float32),
        pltpu.VMEM((RPT,), jnp.float32),
        pltpu.VMEM((RPT,), jnp.float32),
        pltpu.SemaphoreType.DMA,
    ],
)


IBLK = 16
N_IBLK = BPT // IBLK


def _k2_body(nch, ytab, srcp, dstp, out, acc_sp, sidx, didx, rows0, rows1,
             sem0, sem1):
  c = lax.axis_index("c")
  sid = lax.axis_index("s")
  erow0 = sid * BPT
  r0 = sid * RPT

  rows = (rows0, rows1)
  sems = (sem0, sem1)
  for ci in range(nch):
    assigned = (ci * N_CORES) // nch

    @pl.when(c == assigned)
    def _(ci=ci):
      tab = ytab.at[ci]
      pltpu.sync_copy(tab.at[pl.ds(r0, RPT), :], acc_sp.at[pl.ds(r0, RPT), :])
      plsc.subcore_barrier()

      for blk in range(N_IBLK):
        pltpu.sync_copy(srcp.at[pl.ds(erow0 + blk * IBLK, IBLK), :], sidx)
        pltpu.sync_copy(dstp.at[pl.ds(erow0 + blk * IBLK, IBLK), :], didx)
        descs = [None, None]
        descs[0] = pltpu.async_copy(tab.at[sidx.at[0]], rows[0], sems[0])
        for b in range(IBLK):
          if b + 1 < IBLK:
            descs[(b + 1) % 2] = pltpu.async_copy(
                tab.at[sidx.at[b + 1]], rows[(b + 1) % 2], sems[(b + 1) % 2])
          descs[b % 2].wait()
          pltpu.sync_copy(rows[b % 2], acc_sp.at[didx.at[b]], add=True)

      plsc.subcore_barrier()
      pltpu.sync_copy(acc_sp.at[pl.ds(r0, RPT), :],
                      out.at[ci, pl.ds(r0, RPT), :])
      plsc.subcore_barrier()


def _make_k2(nch):
  return pl.kernel(
      functools.partial(_k2_body, nch),
      out_type=jax.ShapeDtypeStruct((nch, N_PAD, LANES), jnp.float32),
      mesh=_MESH,
      scratch_types=[
          pltpu.VMEM_SHARED((N_PAD, LANES), jnp.float32),
          pltpu.VMEM((IBLK, LANES), jnp.int32),
          pltpu.VMEM((IBLK, LANES), jnp.int32),
          pltpu.VMEM((LANES, LANES), jnp.float32),
          pltpu.VMEM((LANES, LANES), jnp.float32),
          pltpu.SemaphoreType.DMA,
          pltpu.SemaphoreType.DMA,
      ],
  )


_k2_2 = _make_k2(2)
_k2_4 = _make_k2(4)


_ROWB = 1024


def _scale_body(x_ref, dis_ref, o_ref):
  o_ref[0] = x_ref[...] * dis_ref[...]


def _k0(xpad, dis2d):
  nch = D_IN // LANES
  return pl.pallas_call(
      _scale_body,
      grid=(nch, N_PAD // _ROWB),
      in_specs=[
          pl.BlockSpec((_ROWB, LANES), lambda ci, rb: (rb, ci)),
          pl.BlockSpec((_ROWB, 1), lambda ci, rb: (rb, 0)),
      ],
      out_specs=pl.BlockSpec((1, _ROWB, LANES), lambda ci, rb: (ci, rb, 0)),
      out_shape=jax.ShapeDtypeStruct((nch, N_PAD, LANES), jnp.float32),
  )(xpad, dis2d)


def _mm_body(g_ref, w_ref, dis_ref, s_ref, b_ref, o_ref, acc_ref,
             *, nk, relu, chunked):
  kc = pl.program_id(2)

  @pl.when(kc == 0)
  def _():
    acc_ref[...] = jnp.zeros_like(acc_ref)

  acc_ref[...] += jnp.dot(g_ref[0], w_ref[...],
                          preferred_element_type=jnp.float32)

  @pl.when(kc == nk - 1)
  def _():
    t = dis_ref[...] * acc_ref[...] + s_ref[...] * b_ref[...]
    if relu:
      t = jnp.maximum(t, 0.0)
      t = dis_ref[...] * t
    if chunked:
      o_ref[0] = t
    else:
      o_ref[...] = t


def _k3(g, W, dis2d, s2d, b2d, relu, chunked):
  nk = g.shape[0]
  nco = D_HID // LANES
  if chunked:
    out_shape = jax.ShapeDtypeStruct((nco, N_PAD, LANES), jnp.float32)
    out_spec = pl.BlockSpec((1, _ROWB, LANES), lambda rb, co, kc: (co, rb, 0))
  else:
    out_shape = jax.ShapeDtypeStruct((N_PAD, D_HID), jnp.float32)
    out_spec = pl.BlockSpec((_ROWB, LANES), lambda rb, co, kc: (rb, co))
  return pl.pallas_call(
      functools.partial(_mm_body, nk=nk, relu=relu, chunked=chunked),
      grid=(N_PAD // _ROWB, nco, nk),
      in_specs=[
          pl.BlockSpec((1, _ROWB, LANES), lambda rb, co, kc: (kc, rb, 0)),
          pl.BlockSpec((LANES, LANES), lambda rb, co, kc: (kc, co)),
          pl.BlockSpec((_ROWB, 1), lambda rb, co, kc: (rb, 0)),
          pl.BlockSpec((_ROWB, 1), lambda rb, co, kc: (rb, 0)),
          pl.BlockSpec((1, LANES), lambda rb, co, kc: (0, co)),
      ],
      out_specs=out_spec,
      out_shape=out_shape,
      scratch_shapes=[pltpu.VMEM((_ROWB, LANES), jnp.float32)],
      compiler_params=pltpu.CompilerParams(
          dimension_semantics=("parallel", "parallel", "arbitrary")),
  )(g, W, dis2d, s2d, b2d)


def kernel(x, edge_index, W1, b1, W2, b2):
  src = edge_index[0].astype(jnp.int32)
  dst = edge_index[1].astype(jnp.int32)
  pad = (jnp.arange(E_PAD - E, dtype=jnp.int32) % N_PAD_ROWS) + N
  srcp = jnp.concatenate([src, pad]).reshape(E_BATCHES, LANES)
  dstp = jnp.concatenate([dst, pad]).reshape(E_BATCHES, LANES)

  dis, s = _k1(srcp, dstp)
  dis2d = dis.reshape(N_PAD, 1)
  s2d = s.reshape(N_PAD, 1)

  xpad = jnp.pad(x, ((0, N_PAD - N), (0, 0)))
  y1 = _k0(xpad, dis2d)
  g1 = _k2_2(y1, srcp, dstp)
  y2 = _k3(g1, W1, dis2d, s2d, b1.reshape(1, D_HID),
           relu=True, chunked=True)
  g2 = _k2_4(y2, srcp, dstp)
  z = _k3(g2, W2, dis2d, s2d, b2.reshape(1, D_HID),
          relu=False, chunked=False)
  return z[:N]
